# baseline (speedup 1.0000x reference)
import functools

import jax
import jax.numpy as jnp
from jax import lax
from jax.experimental import pallas as pl
from jax.experimental.pallas import tpu as pltpu
from jax.experimental.pallas import tpu_sc as plsc

N = 10000
E = 160000
D = 128

NC = 2
NS = 16
NW = NC * NS
P = 5
EP = E // P
PT = EP // NW
CH = 40
PT_CH = PT // CH
PT_PAD = 1008
PER_W = E // NW
PW_PAD = 5008
NP = 10240

_mesh = lambda: plsc.VectorSubcoreMesh(core_axis_name="c", subcore_axis_name="s",
                                       num_cores=NC, num_subcores=NS)


def _wid():
    return lax.axis_index("s") * NC + lax.axis_index("c")


def _gather_body(feat_hbm, fq_hbm, src_hbm, dst_hbm,
                 fu_hbm, fvqd_hbm,
                 src_v, dst_v, srows_v, drows_v, gs0, gs1, ws0, ws1):
    wid = _wid()
    base = wid * PT
    pltpu.sync_copy(src_hbm.at[wid], src_v)
    pltpu.sync_copy(dst_hbm.at[wid], dst_v)
    gsem = (gs0, gs1)
    wsem = (ws0, ws1)

    def do_table(table_hbm, idx_v, rows_v, out_hbm):
        def fire(ci):
            b = ci % 2
            return pltpu.async_copy(table_hbm.at[idx_v.at[ci]],
                                    rows_v.at[b], gsem[b])

        gh = {0: fire(0)}
        wh = {}
        for ci in range(PT_CH):
            b = ci % 2
            if ci + 1 < PT_CH:
                if ci - 1 >= 0:
                    wh[ci - 1].wait()
                gh[ci + 1] = fire(ci + 1)
            gh[ci].wait()
            wh[ci] = pltpu.async_copy(
                rows_v.at[b], out_hbm.at[pl.ds(base + ci * CH, CH)], wsem[b])
        wh[PT_CH - 2].wait()
        wh[PT_CH - 1].wait()

    do_table(feat_hbm, src_v, srows_v, fu_hbm)
    do_table(fq_hbm, dst_v, drows_v, fvqd_hbm)


def _sc_gather(feat, fq, src3, dst3):
    return pl.kernel(
        _gather_body,
        out_type=[jax.ShapeDtypeStruct((EP, D), jnp.float32),
                  jax.ShapeDtypeStruct((EP, 2 * D), jnp.float32)],
        mesh=_mesh(),
        scratch_types=[
            pltpu.VMEM((PT_CH, CH), jnp.int32),
            pltpu.VMEM((PT_CH, CH), jnp.int32),
            pltpu.VMEM((2, CH, D), jnp.float32),
            pltpu.VMEM((2, CH, 2 * D), jnp.float32),
            pltpu.SemaphoreType.DMA,
            pltpu.SemaphoreType.DMA,
            pltpu.SemaphoreType.DMA,
            pltpu.SemaphoreType.DMA,
        ],
    )(feat, fq, src3, dst3)


def _ln_rows(h, w, b):
    m = jnp.mean(h, axis=-1, keepdims=True)
    v = jnp.mean((h - m) ** 2, axis=-1, keepdims=True)
    return (h - m) * jax.lax.rsqrt(v + 1e-5) * w + b


BE = 2000


def _edge_body(fu, fv, qd, kuw, kub, kvw, kvb, kew, keb, knw, knb,
               vuw, vub, vvw, vvb, vew, veb, vnw, vnb, p, g):
    fuv, fvv = fu[...], fv[...]
    hk = jax.nn.gelu(fuv * kuw[...] + kub[...] + fvv * kvw[...] + kvb[...])
    hk = hk * kew[...] + keb[...]
    k = _ln_rows(hk, knw[...], knb[...])
    logit = jnp.sum(k * qd[...], axis=-1, keepdims=True)
    pe = jnp.exp(jnp.minimum(logit, 80.0))
    hv = jax.nn.gelu(fuv * vuw[...] + vub[...] + fvv * vvw[...] + vvb[...])
    hv = hv * vew[...] + veb[...]
    v = _ln_rows(hv, vnw[...], vnb[...])
    p[...] = pe
    g[...] = v * pe


def _tc_edge(pi, fu, fvqd, kuw, kvw, kew, kub, kvb, keb, knw, knb,
             vuw, vvw, vew, vub, vvb, veb, vnw, vnb):
    nb = EP // BE
    row = pl.BlockSpec((BE, D), lambda i: (i, 0))
    fvs = pl.BlockSpec((BE, D), lambda i: (i, 0))
    qds = pl.BlockSpec((BE, D), lambda i: (i, 1))
    wrow = pl.BlockSpec((BE, D), lambda i: (pi * nb + i, 0))
    vec = pl.BlockSpec((1, D), lambda i: (0, 0))
    col = pl.BlockSpec((BE, 1), lambda i: (i, 0))
    return pl.pallas_call(
        _edge_body,
        grid=(EP // BE,),
        in_specs=[row, fvs, qds, wrow, wrow, wrow, wrow, wrow, wrow, vec, vec,
                  wrow, wrow, wrow, wrow, wrow, wrow, vec, vec],
        out_specs=[col, row],
        out_shape=[jax.ShapeDtypeStruct((EP, 1), jnp.float32),
                   jax.ShapeDtypeStruct((EP, D), jnp.float32)],
    )(fu, fvqd, fvqd, kuw, kub, kvw, kvb, kew, keb,
      knw.reshape(1, D), knb.reshape(1, D),
      vuw, vub, vvw, vvb, vew, veb,
      vnw.reshape(1, D), vnb.reshape(1, D))


def _scatter_body(g_hbm, p_hbm, dst2_hbm, dstf_hbm, zn_hbm, znd_hbm,
                  acc_hbm, s_hbm,
                  dst2_v, dstf_v, p_v, rows_v, sloc_v, acc_sh,
                  rs0, rs1, as0, as1):
    rsem = (rs0, rs1)
    asem = (as0, as1)
    cid = lax.axis_index("c")
    sid = lax.axis_index("s")
    wid = sid * NC + cid
    base = wid * PT

    @pl.when(sid == 0)
    def _zero():
        pltpu.sync_copy(znd_hbm, acc_sh)
    plsc.subcore_barrier()

    pltpu.sync_copy(dst2_hbm.at[wid], dst2_v)
    pltpu.sync_copy(zn_hbm, sloc_v)
    dstf_v[pl.ds(PT_PAD - 16, 16)] = jnp.zeros((16,), jnp.int32)
    p_v[pl.ds(PT_PAD - 16, 16)] = jnp.zeros((16,), jnp.float32)
    pltpu.sync_copy(dstf_hbm.at[pl.ds(base, PT)], dstf_v.at[pl.ds(0, PT)])
    pltpu.sync_copy(p_hbm.at[pl.ds(base, PT)], p_v.at[pl.ds(0, PT)])

    def fire_read(ci):
        b = ci % 2
        return pltpu.async_copy(g_hbm.at[pl.ds(base + ci * CH, CH)],
                                rows_v.at[b], rsem[b])

    rh = {0: fire_read(0)}
    ah = {}
    for ci in range(PT_CH):
        b = ci % 2
        if ci + 1 < PT_CH:
            if ci - 1 >= 0:
                ah[ci - 1].wait()
            rh[ci + 1] = fire_read(ci + 1)
        rh[ci].wait()
        ah[ci] = pltpu.async_copy(rows_v.at[b], acc_sh.at[dst2_v.at[ci]],
                                  asem[b], add=True)
    ah[PT_CH - 2].wait()
    ah[PT_CH - 1].wait()

    def sadd(i, carry):
        idx = dstf_v[pl.ds(i * 16, 16)]
        plsc.addupdate_scatter(sloc_v, [idx], p_v[pl.ds(i * 16, 16)])
        return carry
    lax.fori_loop(0, PT_PAD // 16, sadd, 0)

    pltpu.sync_copy(sloc_v, s_hbm.at[wid])
    plsc.subcore_barrier()

    @pl.when(sid < NS - 1)
    def _copy_a():
        pltpu.sync_copy(acc_sh.at[pl.ds(sid * 624, 624)],
                        acc_hbm.at[pl.ds(cid * N + sid * 624, 624)])

    @pl.when(sid == NS - 1)
    def _copy_b():
        pltpu.sync_copy(acc_sh.at[pl.ds(15 * 624, N - 15 * 624)],
                        acc_hbm.at[pl.ds(cid * N + 15 * 624, N - 15 * 624)])


def _sc_scatter(g, p_flat, dst2, dst_flat, zn, znd):
    return pl.kernel(
        _scatter_body,
        out_type=[jax.ShapeDtypeStruct((2 * N, D), jnp.float32),
                  jax.ShapeDtypeStruct((NW, NP), jnp.float32)],
        mesh=_mesh(),
        scratch_types=[
            pltpu.VMEM((PT_CH, CH), jnp.int32),
            pltpu.VMEM((PT_PAD,), jnp.int32),
            pltpu.VMEM((PT_PAD,), jnp.float32),
            pltpu.VMEM((2, CH, D), jnp.float32),
            pltpu.VMEM((NP,), jnp.float32),
            pltpu.VMEM_SHARED((N, D), jnp.float32),
            pltpu.SemaphoreType.DMA,
            pltpu.SemaphoreType.DMA,
            pltpu.SemaphoreType.DMA,
            pltpu.SemaphoreType.DMA,
        ],
        compiler_params=pltpu.CompilerParams(needs_layout_passes=False),
    )(g, p_flat, dst2, dst_flat, zn, znd)


NSL = NP // NS


def _div_body(p_hbm, dstf_hbm, sp0, sp1, sp2, sp3, sp4, attn_hbm, st_hbm,
              dstf_v, p_v, s_v, t_v, a_v, s_red):
    cid = lax.axis_index("c")
    sid = lax.axis_index("s")
    wid = sid * NC + cid
    base = wid * PER_W
    off = sid * NSL

    def zinit(i, c):
        s_v[pl.ds(off + i * 16, 16)] = jnp.zeros((16,), jnp.float32)
        return c
    lax.fori_loop(0, NSL // 16, zinit, 0)

    for sp_hbm in (sp0, sp1, sp2, sp3, sp4):
        pltpu.sync_copy(sp_hbm.at[:, pl.ds(off, NSL)], t_v)

        def slab(t, c):
            def vadd(i, c2):
                s_v[pl.ds(off + i * 16, 16)] = (s_v[pl.ds(off + i * 16, 16)]
                                                + t_v[t, pl.ds(i * 16, 16)])
                return c2
            lax.fori_loop(0, NSL // 16, vadd, 0)
            return c
        lax.fori_loop(0, NW, slab, 0)

    pltpu.sync_copy(s_v.at[pl.ds(off, NSL)], s_red.at[pl.ds(off, NSL)])

    @pl.when(cid == 0)
    def _st_out():
        pltpu.sync_copy(s_v.at[pl.ds(off, NSL)], st_hbm.at[pl.ds(off, NSL)])
    plsc.subcore_barrier()

    pltpu.sync_copy(s_red, s_v)

    def guard(i, c):
        s = s_v[pl.ds(i * 16, 16)]
        s_v[pl.ds(i * 16, 16)] = jnp.where(s == 0.0, 1.0, s)
        return c
    lax.fori_loop(0, NP // 16, guard, 0)

    dstf_v[pl.ds(PW_PAD - 16, 16)] = jnp.zeros((16,), jnp.int32)
    p_v[pl.ds(PW_PAD - 16, 16)] = jnp.zeros((16,), jnp.float32)
    pltpu.sync_copy(dstf_hbm.at[pl.ds(base, PER_W)], dstf_v.at[pl.ds(0, PER_W)])
    pltpu.sync_copy(p_hbm.at[pl.ds(base, PER_W)], p_v.at[pl.ds(0, PER_W)])

    def chunk(i, c):
        idx = dstf_v[pl.ds(i * 16, 16)]
        sv = plsc.load_gather(s_v, [idx])
        a_v[pl.ds(i * 16, 16)] = p_v[pl.ds(i * 16, 16)] / sv
        return c
    lax.fori_loop(0, PW_PAD // 16, chunk, 0)
    pltpu.sync_copy(a_v.at[pl.ds(0, PER_W)], attn_hbm.at[pl.ds(base, PER_W)])


def _sc_div(p_flat, dst_flat, s_parts):
    return pl.kernel(
        _div_body,
        out_type=[jax.ShapeDtypeStruct((E,), jnp.float32),
                  jax.ShapeDtypeStruct((NP,), jnp.float32)],
        mesh=_mesh(),
        scratch_types=[
            pltpu.VMEM((PW_PAD,), jnp.int32),
            pltpu.VMEM((PW_PAD,), jnp.float32),
            pltpu.VMEM((NP,), jnp.float32),
            pltpu.VMEM((NW, NSL), jnp.float32),
            pltpu.VMEM((PW_PAD,), jnp.float32),
            pltpu.VMEM_SHARED((NP,), jnp.float32),
        ],
        compiler_params=pltpu.CompilerParams(needs_layout_passes=False),
    )(p_flat, dst_flat, *s_parts)


BN = 400


def _node_body(p0, p1, p2, p3, p4, p5, p6, p7, p8, p9,
               st, w0, b0, w1, b1, nw, nb, out):
    num = (p0[...] + p1[...] + p2[...] + p3[...] + p4[...]
           + p5[...] + p6[...] + p7[...] + p8[...] + p9[...])
    s = st[...]
    nf = num / jnp.where(s == 0.0, 1.0, s)
    h = jax.nn.gelu(nf * w0[...] + b0[...])
    h = h * w1[...] + b1[...]
    out[...] = _ln_rows(h, nw[...], nb[...])


def _tc_node(parts, st, w0, b0, w1, b1, nw, nb):
    row = pl.BlockSpec((BN, D), lambda i: (i, 0))
    col = pl.BlockSpec((BN, 1), lambda i: (i, 0))
    vec = pl.BlockSpec((1, D), lambda i: (0, 0))
    return pl.pallas_call(
        _node_body,
        grid=(N // BN,),
        in_specs=[row] * 10 + [col, row, row, row, row, vec, vec],
        out_specs=row,
        out_shape=jax.ShapeDtypeStruct((N, D), jnp.float32),
    )(*parts, st, w0, b0, w1, b1, nw.reshape(1, D), nb.reshape(1, D))


def kernel(feat, query, edge_index, src_key_w, dst_key_w, edge_key_w,
           src_key_b, dst_key_b, edge_key_b, src_val_w, dst_val_w, edge_val_w,
           src_val_b, dst_val_b, edge_val_b, node_weight, node_bias,
           key_norm_w, key_norm_b, value_norm_w, value_norm_b,
           node_norm_w, node_norm_b):
    src = edge_index[0].astype(jnp.int32)
    dst = edge_index[1].astype(jnp.int32)
    src4 = src.reshape(P, NW, PT_CH, CH)
    dst4 = dst.reshape(P, NW, PT_CH, CH)

    zn = jnp.zeros((NP,), jnp.float32)
    znd = jnp.zeros((N, D), jnp.float32)

    fq = jnp.concatenate([feat, query], axis=1)
    gathered = [_sc_gather(feat, fq, src4[i], dst4[i]) for i in range(P)]

    ps, acc_views, sps = [], [], []
    for i in range(P):
        fu, fvqd = gathered[i]
        p, g = _tc_edge(i, fu, fvqd,
                        src_key_w, dst_key_w, edge_key_w,
                        src_key_b, dst_key_b, edge_key_b,
                        key_norm_w, key_norm_b,
                        src_val_w, dst_val_w, edge_val_w,
                        src_val_b, dst_val_b, edge_val_b,
                        value_norm_w, value_norm_b)
        p_flat = p.reshape(EP)
        acc, sp = _sc_scatter(g, p_flat, dst4[i], dst[i * EP:(i + 1) * EP],
                              zn, znd)
        ps.append(p_flat)
        acc_views.extend([acc[:N], acc[N:]])
        sps.append(sp)

    p_all = jnp.concatenate(ps)
    attn, s_total = _sc_div(p_all, dst, sps)

    out = _tc_node(acc_views, s_total[:N].reshape(N, 1),
                   node_weight[:, 0], node_bias[:, 0],
                   node_weight[:, 1], node_bias[:, 1],
                   node_norm_w, node_norm_b)
    return out, attn.reshape(E, 1)


# speedup vs baseline: 1.0098x; 1.0098x over previous
import functools

import jax
import jax.numpy as jnp
from jax import lax
from jax.experimental import pallas as pl
from jax.experimental.pallas import tpu as pltpu
from jax.experimental.pallas import tpu_sc as plsc

N = 10000
E = 160000
D = 128

NC = 2
NS = 16
NW = NC * NS
P = 5
EP = E // P
PT = EP // NW
CH = 40
NBUF = 4
PT_CH = PT // CH
PT_PAD = 1008
PER_W = E // NW
PW_PAD = 5008
NP = 10240

_mesh = lambda: plsc.VectorSubcoreMesh(core_axis_name="c", subcore_axis_name="s",
                                       num_cores=NC, num_subcores=NS)


def _wid():
    return lax.axis_index("s") * NC + lax.axis_index("c")


def _gather_body(feat_hbm, fq_hbm, src_hbm, dst_hbm,
                 fu_hbm, fvqd_hbm,
                 src_v, dst_v, srows_v, drows_v,
                 gs0, gs1, gs2, gs3, ws0, ws1, ws2, ws3):
    wid = _wid()
    base = wid * PT
    pltpu.sync_copy(src_hbm.at[wid], src_v)
    pltpu.sync_copy(dst_hbm.at[wid], dst_v)
    gsem = (gs0, gs1, gs2, gs3)
    wsem = (ws0, ws1, ws2, ws3)

    def do_table(table_hbm, idx_v, rows_v, out_hbm):
        def fire(ci):
            b = ci % NBUF
            return pltpu.async_copy(table_hbm.at[idx_v.at[ci]],
                                    rows_v.at[b], gsem[b])

        gh, wh = {}, {}
        for c in range(min(NBUF - 1, PT_CH)):
            gh[c] = fire(c)
        for ci in range(PT_CH):
            b = ci % NBUF
            nxt = ci + NBUF - 1
            if nxt < PT_CH:
                if nxt - NBUF >= 0:
                    wh[nxt - NBUF].wait()
                gh[nxt] = fire(nxt)
            gh[ci].wait()
            wh[ci] = pltpu.async_copy(
                rows_v.at[b], out_hbm.at[pl.ds(base + ci * CH, CH)], wsem[b])
        for k in range(PT_CH - NBUF, PT_CH):
            wh[k].wait()

    do_table(feat_hbm, src_v, srows_v, fu_hbm)
    do_table(fq_hbm, dst_v, drows_v, fvqd_hbm)


def _sc_gather(feat, fq, src3, dst3):
    return pl.kernel(
        _gather_body,
        out_type=[jax.ShapeDtypeStruct((EP, D), jnp.float32),
                  jax.ShapeDtypeStruct((EP, 2 * D), jnp.float32)],
        mesh=_mesh(),
        scratch_types=[
            pltpu.VMEM((PT_CH, CH), jnp.int32),
            pltpu.VMEM((PT_CH, CH), jnp.int32),
            pltpu.VMEM((NBUF, CH, D), jnp.float32),
            pltpu.VMEM((NBUF, CH, 2 * D), jnp.float32),
        ] + [pltpu.SemaphoreType.DMA] * (2 * NBUF),
    )(feat, fq, src3, dst3)


def _ln_rows(h, w, b):
    m = jnp.mean(h, axis=-1, keepdims=True)
    v = jnp.mean((h - m) ** 2, axis=-1, keepdims=True)
    return (h - m) * jax.lax.rsqrt(v + 1e-5) * w + b


BE = 2000


def _edge_body(fu, fv, qd, kuw, kub, kvw, kvb, kew, keb, knw, knb,
               vuw, vub, vvw, vvb, vew, veb, vnw, vnb, p, g):
    fuv, fvv = fu[...], fv[...]
    hk = jax.nn.gelu(fuv * kuw[...] + kub[...] + fvv * kvw[...] + kvb[...])
    hk = hk * kew[...] + keb[...]
    k = _ln_rows(hk, knw[...], knb[...])
    logit = jnp.sum(k * qd[...], axis=-1, keepdims=True)
    pe = jnp.exp(jnp.minimum(logit, 80.0))
    hv = jax.nn.gelu(fuv * vuw[...] + vub[...] + fvv * vvw[...] + vvb[...])
    hv = hv * vew[...] + veb[...]
    v = _ln_rows(hv, vnw[...], vnb[...])
    p[...] = pe
    g[...] = v * pe


def _tc_edge(pi, fu, fvqd, kuw, kvw, kew, kub, kvb, keb, knw, knb,
             vuw, vvw, vew, vub, vvb, veb, vnw, vnb):
    nb = EP // BE
    row = pl.BlockSpec((BE, D), lambda i: (i, 0))
    fvs = pl.BlockSpec((BE, D), lambda i: (i, 0))
    qds = pl.BlockSpec((BE, D), lambda i: (i, 1))
    wrow = pl.BlockSpec((BE, D), lambda i: (pi * nb + i, 0))
    vec = pl.BlockSpec((1, D), lambda i: (0, 0))
    col = pl.BlockSpec((BE, 1), lambda i: (i, 0))
    return pl.pallas_call(
        _edge_body,
        grid=(EP // BE,),
        in_specs=[row, fvs, qds, wrow, wrow, wrow, wrow, wrow, wrow, vec, vec,
                  wrow, wrow, wrow, wrow, wrow, wrow, vec, vec],
        out_specs=[col, row],
        out_shape=[jax.ShapeDtypeStruct((EP, 1), jnp.float32),
                   jax.ShapeDtypeStruct((EP, D), jnp.float32)],
    )(fu, fvqd, fvqd, kuw, kub, kvw, kvb, kew, keb,
      knw.reshape(1, D), knb.reshape(1, D),
      vuw, vub, vvw, vvb, vew, veb,
      vnw.reshape(1, D), vnb.reshape(1, D))


def _scatter_body(g_hbm, p_hbm, dst2_hbm, dstf_hbm, zn_hbm, znd_hbm,
                  acc_hbm, s_hbm,
                  dst2_v, dstf_v, p_v, rows_v, sloc_v, acc_sh,
                  rs0, rs1, rs2, rs3, as0, as1, as2, as3):
    rsem = (rs0, rs1, rs2, rs3)
    asem = (as0, as1, as2, as3)
    cid = lax.axis_index("c")
    sid = lax.axis_index("s")
    wid = sid * NC + cid
    base = wid * PT

    @pl.when(sid == 0)
    def _zero():
        pltpu.sync_copy(znd_hbm, acc_sh)
    plsc.subcore_barrier()

    pltpu.sync_copy(dst2_hbm.at[wid], dst2_v)
    pltpu.sync_copy(zn_hbm, sloc_v)
    dstf_v[pl.ds(PT_PAD - 16, 16)] = jnp.zeros((16,), jnp.int32)
    p_v[pl.ds(PT_PAD - 16, 16)] = jnp.zeros((16,), jnp.float32)
    pltpu.sync_copy(dstf_hbm.at[pl.ds(base, PT)], dstf_v.at[pl.ds(0, PT)])
    pltpu.sync_copy(p_hbm.at[pl.ds(base, PT)], p_v.at[pl.ds(0, PT)])

    def fire_read(ci):
        b = ci % NBUF
        return pltpu.async_copy(g_hbm.at[pl.ds(base + ci * CH, CH)],
                                rows_v.at[b], rsem[b])

    rh, ah = {}, {}
    for c in range(min(NBUF - 1, PT_CH)):
        rh[c] = fire_read(c)
    for ci in range(PT_CH):
        b = ci % NBUF
        nxt = ci + NBUF - 1
        if nxt < PT_CH:
            if nxt - NBUF >= 0:
                ah[nxt - NBUF].wait()
            rh[nxt] = fire_read(nxt)
        rh[ci].wait()
        ah[ci] = pltpu.async_copy(rows_v.at[b], acc_sh.at[dst2_v.at[ci]],
                                  asem[b], add=True)
    for k in range(PT_CH - NBUF, PT_CH):
        ah[k].wait()

    def sadd(i, carry):
        idx = dstf_v[pl.ds(i * 16, 16)]
        plsc.addupdate_scatter(sloc_v, [idx], p_v[pl.ds(i * 16, 16)])
        return carry
    lax.fori_loop(0, PT_PAD // 16, sadd, 0)

    pltpu.sync_copy(sloc_v, s_hbm.at[wid])
    plsc.subcore_barrier()

    @pl.when(sid < NS - 1)
    def _copy_a():
        pltpu.sync_copy(acc_sh.at[pl.ds(sid * 624, 624)],
                        acc_hbm.at[pl.ds(cid * N + sid * 624, 624)])

    @pl.when(sid == NS - 1)
    def _copy_b():
        pltpu.sync_copy(acc_sh.at[pl.ds(15 * 624, N - 15 * 624)],
                        acc_hbm.at[pl.ds(cid * N + 15 * 624, N - 15 * 624)])


def _sc_scatter(g, p_flat, dst2, dst_flat, zn, znd):
    return pl.kernel(
        _scatter_body,
        out_type=[jax.ShapeDtypeStruct((2 * N, D), jnp.float32),
                  jax.ShapeDtypeStruct((NW, NP), jnp.float32)],
        mesh=_mesh(),
        scratch_types=[
            pltpu.VMEM((PT_CH, CH), jnp.int32),
            pltpu.VMEM((PT_PAD,), jnp.int32),
            pltpu.VMEM((PT_PAD,), jnp.float32),
            pltpu.VMEM((NBUF, CH, D), jnp.float32),
            pltpu.VMEM((NP,), jnp.float32),
            pltpu.VMEM_SHARED((N, D), jnp.float32),
        ] + [pltpu.SemaphoreType.DMA] * (2 * NBUF),
        compiler_params=pltpu.CompilerParams(needs_layout_passes=False),
    )(g, p_flat, dst2, dst_flat, zn, znd)


NSL = NP // NS


def _div_body(p_hbm, dstf_hbm, sp0, sp1, sp2, sp3, sp4, attn_hbm, st_hbm,
              dstf_v, p_v, s_v, t_v, a_v, s_red):
    cid = lax.axis_index("c")
    sid = lax.axis_index("s")
    wid = sid * NC + cid
    base = wid * PER_W
    off = sid * NSL

    def zinit(i, c):
        s_v[pl.ds(off + i * 16, 16)] = jnp.zeros((16,), jnp.float32)
        return c
    lax.fori_loop(0, NSL // 16, zinit, 0)

    for sp_hbm in (sp0, sp1, sp2, sp3, sp4):
        pltpu.sync_copy(sp_hbm.at[:, pl.ds(off, NSL)], t_v)

        def slab(t, c):
            def vadd(i, c2):
                s_v[pl.ds(off + i * 16, 16)] = (s_v[pl.ds(off + i * 16, 16)]
                                                + t_v[t, pl.ds(i * 16, 16)])
                return c2
            lax.fori_loop(0, NSL // 16, vadd, 0)
            return c
        lax.fori_loop(0, NW, slab, 0)

    pltpu.sync_copy(s_v.at[pl.ds(off, NSL)], s_red.at[pl.ds(off, NSL)])

    @pl.when(cid == 0)
    def _st_out():
        pltpu.sync_copy(s_v.at[pl.ds(off, NSL)], st_hbm.at[pl.ds(off, NSL)])
    plsc.subcore_barrier()

    pltpu.sync_copy(s_red, s_v)

    def guard(i, c):
        s = s_v[pl.ds(i * 16, 16)]
        s_v[pl.ds(i * 16, 16)] = jnp.where(s == 0.0, 1.0, s)
        return c
    lax.fori_loop(0, NP // 16, guard, 0)

    dstf_v[pl.ds(PW_PAD - 16, 16)] = jnp.zeros((16,), jnp.int32)
    p_v[pl.ds(PW_PAD - 16, 16)] = jnp.zeros((16,), jnp.float32)
    pltpu.sync_copy(dstf_hbm.at[pl.ds(base, PER_W)], dstf_v.at[pl.ds(0, PER_W)])
    pltpu.sync_copy(p_hbm.at[pl.ds(base, PER_W)], p_v.at[pl.ds(0, PER_W)])

    def chunk(i, c):
        idx = dstf_v[pl.ds(i * 16, 16)]
        sv = plsc.load_gather(s_v, [idx])
        a_v[pl.ds(i * 16, 16)] = p_v[pl.ds(i * 16, 16)] / sv
        return c
    lax.fori_loop(0, PW_PAD // 16, chunk, 0)
    pltpu.sync_copy(a_v.at[pl.ds(0, PER_W)], attn_hbm.at[pl.ds(base, PER_W)])


def _sc_div(p_flat, dst_flat, s_parts):
    return pl.kernel(
        _div_body,
        out_type=[jax.ShapeDtypeStruct((E,), jnp.float32),
                  jax.ShapeDtypeStruct((NP,), jnp.float32)],
        mesh=_mesh(),
        scratch_types=[
            pltpu.VMEM((PW_PAD,), jnp.int32),
            pltpu.VMEM((PW_PAD,), jnp.float32),
            pltpu.VMEM((NP,), jnp.float32),
            pltpu.VMEM((NW, NSL), jnp.float32),
            pltpu.VMEM((PW_PAD,), jnp.float32),
            pltpu.VMEM_SHARED((NP,), jnp.float32),
        ],
        compiler_params=pltpu.CompilerParams(needs_layout_passes=False),
    )(p_flat, dst_flat, *s_parts)


BN = 400


def _node_body(p0, p1, p2, p3, p4, p5, p6, p7, p8, p9,
               st, w0, b0, w1, b1, nw, nb, out):
    num = (p0[...] + p1[...] + p2[...] + p3[...] + p4[...]
           + p5[...] + p6[...] + p7[...] + p8[...] + p9[...])
    s = st[...]
    nf = num / jnp.where(s == 0.0, 1.0, s)
    h = jax.nn.gelu(nf * w0[...] + b0[...])
    h = h * w1[...] + b1[...]
    out[...] = _ln_rows(h, nw[...], nb[...])


def _tc_node(parts, st, w0, b0, w1, b1, nw, nb):
    row = pl.BlockSpec((BN, D), lambda i: (i, 0))
    col = pl.BlockSpec((BN, 1), lambda i: (i, 0))
    vec = pl.BlockSpec((1, D), lambda i: (0, 0))
    return pl.pallas_call(
        _node_body,
        grid=(N // BN,),
        in_specs=[row] * 10 + [col, row, row, row, row, vec, vec],
        out_specs=row,
        out_shape=jax.ShapeDtypeStruct((N, D), jnp.float32),
    )(*parts, st, w0, b0, w1, b1, nw.reshape(1, D), nb.reshape(1, D))


def kernel(feat, query, edge_index, src_key_w, dst_key_w, edge_key_w,
           src_key_b, dst_key_b, edge_key_b, src_val_w, dst_val_w, edge_val_w,
           src_val_b, dst_val_b, edge_val_b, node_weight, node_bias,
           key_norm_w, key_norm_b, value_norm_w, value_norm_b,
           node_norm_w, node_norm_b):
    src = edge_index[0].astype(jnp.int32)
    dst = edge_index[1].astype(jnp.int32)
    src4 = src.reshape(P, NW, PT_CH, CH)
    dst4 = dst.reshape(P, NW, PT_CH, CH)

    zn = jnp.zeros((NP,), jnp.float32)
    znd = jnp.zeros((N, D), jnp.float32)

    fq = jnp.concatenate([feat, query], axis=1)
    gathered = [_sc_gather(feat, fq, src4[i], dst4[i]) for i in range(P)]

    ps, acc_views, sps = [], [], []
    for i in range(P):
        fu, fvqd = gathered[i]
        p, g = _tc_edge(i, fu, fvqd,
                        src_key_w, dst_key_w, edge_key_w,
                        src_key_b, dst_key_b, edge_key_b,
                        key_norm_w, key_norm_b,
                        src_val_w, dst_val_w, edge_val_w,
                        src_val_b, dst_val_b, edge_val_b,
                        value_norm_w, value_norm_b)
        p_flat = p.reshape(EP)
        acc, sp = _sc_scatter(g, p_flat, dst4[i], dst[i * EP:(i + 1) * EP],
                              zn, znd)
        ps.append(p_flat)
        acc_views.extend([acc[:N], acc[N:]])
        sps.append(sp)

    p_all = jnp.concatenate(ps)
    attn, s_total = _sc_div(p_all, dst, sps)

    out = _tc_node(acc_views, s_total[:N].reshape(N, 1),
                   node_weight[:, 0], node_bias[:, 0],
                   node_weight[:, 1], node_bias[:, 1],
                   node_norm_w, node_norm_b)
    return out, attn.reshape(E, 1)


# speedup vs baseline: 1.0200x; 1.0101x over previous
import functools

import jax
import jax.numpy as jnp
from jax import lax
from jax.experimental import pallas as pl
from jax.experimental.pallas import tpu as pltpu
from jax.experimental.pallas import tpu_sc as plsc

N = 10000
E = 160000
D = 128

NC = 2
NS = 16
NW = NC * NS
PARTS = (12800, 38400, 44800, 44800, 19200)
P = len(PARTS)
OFFS = tuple(sum(PARTS[:i]) for i in range(P))
CH = 40
NBUF = 4
PER_W = E // NW
PW_PAD = 5008
NP = 10240


def _pad16(n):
    return ((n + 15) // 16) * 16

_mesh = lambda: plsc.VectorSubcoreMesh(core_axis_name="c", subcore_axis_name="s",
                                       num_cores=NC, num_subcores=NS)


def _wid():
    return lax.axis_index("s") * NC + lax.axis_index("c")


def _make_gather_body(pt):
    pt_ch = pt // CH

    def body(feat_hbm, fq_hbm, src_hbm, dst_hbm,
             fu_hbm, fvqd_hbm,
             src_v, dst_v, srows_v, drows_v,
             gs0, gs1, gs2, gs3, ws0, ws1, ws2, ws3):
        wid = _wid()
        base = wid * pt
        pltpu.sync_copy(src_hbm.at[wid], src_v)
        pltpu.sync_copy(dst_hbm.at[wid], dst_v)
        gsem = (gs0, gs1, gs2, gs3)
        wsem = (ws0, ws1, ws2, ws3)

        def do_table(table_hbm, idx_v, rows_v, out_hbm):
            def fire(ci):
                b = ci % NBUF
                return pltpu.async_copy(table_hbm.at[idx_v.at[ci]],
                                        rows_v.at[b], gsem[b])

            gh, wh = {}, {}
            for c in range(min(NBUF - 1, pt_ch)):
                gh[c] = fire(c)
            for ci in range(pt_ch):
                b = ci % NBUF
                nxt = ci + NBUF - 1
                if nxt < pt_ch:
                    if nxt - NBUF >= 0:
                        wh[nxt - NBUF].wait()
                    gh[nxt] = fire(nxt)
                gh[ci].wait()
                wh[ci] = pltpu.async_copy(
                    rows_v.at[b], out_hbm.at[pl.ds(base + ci * CH, CH)],
                    wsem[b])
            for k in range(max(0, pt_ch - NBUF), pt_ch):
                wh[k].wait()

        do_table(feat_hbm, src_v, srows_v, fu_hbm)
        do_table(fq_hbm, dst_v, drows_v, fvqd_hbm)

    return body


def _sc_gather(feat, fq, src3, dst3, ep):
    pt = ep // NW
    return pl.kernel(
        _make_gather_body(pt),
        out_type=[jax.ShapeDtypeStruct((ep, D), jnp.float32),
                  jax.ShapeDtypeStruct((ep, 2 * D), jnp.float32)],
        mesh=_mesh(),
        scratch_types=[
            pltpu.VMEM((pt // CH, CH), jnp.int32),
            pltpu.VMEM((pt // CH, CH), jnp.int32),
            pltpu.VMEM((NBUF, CH, D), jnp.float32),
            pltpu.VMEM((NBUF, CH, 2 * D), jnp.float32),
        ] + [pltpu.SemaphoreType.DMA] * (2 * NBUF),
    )(feat, fq, src3, dst3)


def _ln_rows(h, w, b):
    m = jnp.mean(h, axis=-1, keepdims=True)
    v = jnp.mean((h - m) ** 2, axis=-1, keepdims=True)
    return (h - m) * jax.lax.rsqrt(v + 1e-5) * w + b


BE = 1600


def _edge_body(fu, fv, qd, kuw, kub, kvw, kvb, kew, keb, knw, knb,
               vuw, vub, vvw, vvb, vew, veb, vnw, vnb, p, g):
    fuv, fvv = fu[...], fv[...]
    hk = jax.nn.gelu(fuv * kuw[...] + kub[...] + fvv * kvw[...] + kvb[...])
    hk = hk * kew[...] + keb[...]
    k = _ln_rows(hk, knw[...], knb[...])
    logit = jnp.sum(k * qd[...], axis=-1, keepdims=True)
    pe = jnp.exp(jnp.minimum(logit, 80.0))
    hv = jax.nn.gelu(fuv * vuw[...] + vub[...] + fvv * vvw[...] + vvb[...])
    hv = hv * vew[...] + veb[...]
    v = _ln_rows(hv, vnw[...], vnb[...])
    p[...] = pe
    g[...] = v * pe


def _tc_edge(off, ep, fu, fvqd, kuw, kvw, kew, kub, kvb, keb, knw, knb,
             vuw, vvw, vew, vub, vvb, veb, vnw, vnb):
    ob = off // BE
    row = pl.BlockSpec((BE, D), lambda i: (i, 0))
    fvs = pl.BlockSpec((BE, D), lambda i: (i, 0))
    qds = pl.BlockSpec((BE, D), lambda i: (i, 1))
    wrow = pl.BlockSpec((BE, D), lambda i: (ob + i, 0))
    vec = pl.BlockSpec((1, D), lambda i: (0, 0))
    col = pl.BlockSpec((BE, 1), lambda i: (i, 0))
    return pl.pallas_call(
        _edge_body,
        grid=(ep // BE,),
        in_specs=[row, fvs, qds, wrow, wrow, wrow, wrow, wrow, wrow, vec, vec,
                  wrow, wrow, wrow, wrow, wrow, wrow, vec, vec],
        out_specs=[col, row],
        out_shape=[jax.ShapeDtypeStruct((ep, 1), jnp.float32),
                   jax.ShapeDtypeStruct((ep, D), jnp.float32)],
    )(fu, fvqd, fvqd, kuw, kub, kvw, kvb, kew, keb,
      knw.reshape(1, D), knb.reshape(1, D),
      vuw, vub, vvw, vvb, vew, veb,
      vnw.reshape(1, D), vnb.reshape(1, D))


def _make_scatter_body(pt):
    pt_ch = pt // CH
    pt_pad = _pad16(pt)

    def body(g_hbm, p_hbm, dst2_hbm, dstf_hbm, zn_hbm, znd_hbm,
             acc_hbm, s_hbm,
             dst2_v, dstf_v, p_v, rows_v, sloc_v, acc_sh,
             rs0, rs1, rs2, rs3, as0, as1, as2, as3):
        rsem = (rs0, rs1, rs2, rs3)
        asem = (as0, as1, as2, as3)
        cid = lax.axis_index("c")
        sid = lax.axis_index("s")
        wid = sid * NC + cid
        base = wid * pt

        @pl.when(sid == 0)
        def _zero():
            pltpu.sync_copy(znd_hbm, acc_sh)
        plsc.subcore_barrier()

        pltpu.sync_copy(dst2_hbm.at[wid], dst2_v)
        pltpu.sync_copy(zn_hbm, sloc_v)
        dstf_v[pl.ds(pt_pad - 16, 16)] = jnp.zeros((16,), jnp.int32)
        p_v[pl.ds(pt_pad - 16, 16)] = jnp.zeros((16,), jnp.float32)
        pltpu.sync_copy(dstf_hbm.at[pl.ds(base, pt)], dstf_v.at[pl.ds(0, pt)])
        pltpu.sync_copy(p_hbm.at[pl.ds(base, pt)], p_v.at[pl.ds(0, pt)])

        def fire_read(ci):
            b = ci % NBUF
            return pltpu.async_copy(g_hbm.at[pl.ds(base + ci * CH, CH)],
                                    rows_v.at[b], rsem[b])

        rh, ah = {}, {}
        for c in range(min(NBUF - 1, pt_ch)):
            rh[c] = fire_read(c)
        for ci in range(pt_ch):
            b = ci % NBUF
            nxt = ci + NBUF - 1
            if nxt < pt_ch:
                if nxt - NBUF >= 0:
                    ah[nxt - NBUF].wait()
                rh[nxt] = fire_read(nxt)
            rh[ci].wait()
            ah[ci] = pltpu.async_copy(rows_v.at[b], acc_sh.at[dst2_v.at[ci]],
                                      asem[b], add=True)
        for k in range(max(0, pt_ch - NBUF), pt_ch):
            ah[k].wait()

        def sadd(i, carry):
            idx = dstf_v[pl.ds(i * 16, 16)]
            plsc.addupdate_scatter(sloc_v, [idx], p_v[pl.ds(i * 16, 16)])
            return carry
        lax.fori_loop(0, pt_pad // 16, sadd, 0)

        pltpu.sync_copy(sloc_v, s_hbm.at[wid])
        plsc.subcore_barrier()

        @pl.when(sid < NS - 1)
        def _copy_a():
            pltpu.sync_copy(acc_sh.at[pl.ds(sid * 624, 624)],
                            acc_hbm.at[pl.ds(cid * N + sid * 624, 624)])

        @pl.when(sid == NS - 1)
        def _copy_b():
            pltpu.sync_copy(acc_sh.at[pl.ds(15 * 624, N - 15 * 624)],
                            acc_hbm.at[pl.ds(cid * N + 15 * 624,
                                             N - 15 * 624)])

    return body


def _sc_scatter(g, p_flat, dst2, dst_flat, zn, znd, ep):
    pt = ep // NW
    pt_pad = _pad16(pt)
    return pl.kernel(
        _make_scatter_body(pt),
        out_type=[jax.ShapeDtypeStruct((2 * N, D), jnp.float32),
                  jax.ShapeDtypeStruct((NW, NP), jnp.float32)],
        mesh=_mesh(),
        scratch_types=[
            pltpu.VMEM((pt // CH, CH), jnp.int32),
            pltpu.VMEM((pt_pad,), jnp.int32),
            pltpu.VMEM((pt_pad,), jnp.float32),
            pltpu.VMEM((NBUF, CH, D), jnp.float32),
            pltpu.VMEM((NP,), jnp.float32),
            pltpu.VMEM_SHARED((N, D), jnp.float32),
        ] + [pltpu.SemaphoreType.DMA] * (2 * NBUF),
        compiler_params=pltpu.CompilerParams(needs_layout_passes=False),
    )(g, p_flat, dst2, dst_flat, zn, znd)


NSL = NP // NS


def _div_body(p_hbm, dstf_hbm, sp0, sp1, sp2, sp3, sp4, attn_hbm,
              dstf_v, p_v, s_v, t_v, a_v, s_red):
    cid = lax.axis_index("c")
    sid = lax.axis_index("s")
    wid = sid * NC + cid
    base = wid * PER_W
    off = sid * NSL

    def zinit(i, c):
        s_v[pl.ds(off + i * 16, 16)] = jnp.zeros((16,), jnp.float32)
        return c
    lax.fori_loop(0, NSL // 16, zinit, 0)

    for sp_hbm in (sp0, sp1, sp2, sp3, sp4):
        pltpu.sync_copy(sp_hbm.at[:, pl.ds(off, NSL)], t_v)

        def slab(t, c):
            for i in range(NSL // 16):
                s_v[pl.ds(off + i * 16, 16)] = (s_v[pl.ds(off + i * 16, 16)]
                                                + t_v[t, pl.ds(i * 16, 16)])
            return c
        lax.fori_loop(0, NW, slab, 0)

    pltpu.sync_copy(s_v.at[pl.ds(off, NSL)], s_red.at[pl.ds(off, NSL)])
    plsc.subcore_barrier()

    pltpu.sync_copy(s_red, s_v)

    def guard(i, c):
        s = s_v[pl.ds(i * 16, 16)]
        s_v[pl.ds(i * 16, 16)] = jnp.where(s == 0.0, 1.0, s)
        return c
    lax.fori_loop(0, NP // 16, guard, 0)

    dstf_v[pl.ds(PW_PAD - 16, 16)] = jnp.zeros((16,), jnp.int32)
    p_v[pl.ds(PW_PAD - 16, 16)] = jnp.zeros((16,), jnp.float32)
    pltpu.sync_copy(dstf_hbm.at[pl.ds(base, PER_W)], dstf_v.at[pl.ds(0, PER_W)])
    pltpu.sync_copy(p_hbm.at[pl.ds(base, PER_W)], p_v.at[pl.ds(0, PER_W)])

    def chunk(i, c):
        idx = dstf_v[pl.ds(i * 16, 16)]
        sv = plsc.load_gather(s_v, [idx])
        a_v[pl.ds(i * 16, 16)] = p_v[pl.ds(i * 16, 16)] / sv
        return c
    lax.fori_loop(0, PW_PAD // 16, chunk, 0)
    pltpu.sync_copy(a_v.at[pl.ds(0, PER_W)], attn_hbm.at[pl.ds(base, PER_W)])


def _sc_div(p_flat, dst_flat, s_parts):
    return pl.kernel(
        _div_body,
        out_type=jax.ShapeDtypeStruct((E,), jnp.float32),
        mesh=_mesh(),
        scratch_types=[
            pltpu.VMEM((PW_PAD,), jnp.int32),
            pltpu.VMEM((PW_PAD,), jnp.float32),
            pltpu.VMEM((NP,), jnp.float32),
            pltpu.VMEM((NW, NSL), jnp.float32),
            pltpu.VMEM((PW_PAD,), jnp.float32),
            pltpu.VMEM_SHARED((NP,), jnp.float32),
        ],
        compiler_params=pltpu.CompilerParams(needs_layout_passes=False),
    )(p_flat, dst_flat, *s_parts)


def _sred_body(s0, s1, s2, s3, s4, o):
    o[...] = (jnp.sum(s0[...], axis=0, keepdims=True)
              + jnp.sum(s1[...], axis=0, keepdims=True)
              + jnp.sum(s2[...], axis=0, keepdims=True)
              + jnp.sum(s3[...], axis=0, keepdims=True)
              + jnp.sum(s4[...], axis=0, keepdims=True))


def _tc_sred(sps):
    lb = 1024
    blk = pl.BlockSpec((NW, lb), lambda i: (0, i))
    return pl.pallas_call(
        _sred_body,
        grid=(NP // lb,),
        in_specs=[blk] * 5,
        out_specs=pl.BlockSpec((1, lb), lambda i: (0, i)),
        out_shape=jax.ShapeDtypeStruct((1, NP), jnp.float32),
    )(*sps)


BN = 400


def _node_body(p0, p1, p2, p3, p4, p5, p6, p7, p8, p9,
               st, w0, b0, w1, b1, nw, nb, out):
    num = (p0[...] + p1[...] + p2[...] + p3[...] + p4[...]
           + p5[...] + p6[...] + p7[...] + p8[...] + p9[...])
    s = st[...]
    nf = num / jnp.where(s == 0.0, 1.0, s)
    h = jax.nn.gelu(nf * w0[...] + b0[...])
    h = h * w1[...] + b1[...]
    out[...] = _ln_rows(h, nw[...], nb[...])


def _tc_node(parts, st, w0, b0, w1, b1, nw, nb):
    row = pl.BlockSpec((BN, D), lambda i: (i, 0))
    col = pl.BlockSpec((BN, 1), lambda i: (i, 0))
    vec = pl.BlockSpec((1, D), lambda i: (0, 0))
    return pl.pallas_call(
        _node_body,
        grid=(N // BN,),
        in_specs=[row] * 10 + [col, row, row, row, row, vec, vec],
        out_specs=row,
        out_shape=jax.ShapeDtypeStruct((N, D), jnp.float32),
    )(*parts, st, w0, b0, w1, b1, nw.reshape(1, D), nb.reshape(1, D))


def kernel(feat, query, edge_index, src_key_w, dst_key_w, edge_key_w,
           src_key_b, dst_key_b, edge_key_b, src_val_w, dst_val_w, edge_val_w,
           src_val_b, dst_val_b, edge_val_b, node_weight, node_bias,
           key_norm_w, key_norm_b, value_norm_w, value_norm_b,
           node_norm_w, node_norm_b):
    src = edge_index[0].astype(jnp.int32)
    dst = edge_index[1].astype(jnp.int32)
    srcp = [src[OFFS[i]:OFFS[i] + PARTS[i]].reshape(NW, PARTS[i] // NW // CH, CH)
            for i in range(P)]
    dstp = [dst[OFFS[i]:OFFS[i] + PARTS[i]].reshape(NW, PARTS[i] // NW // CH, CH)
            for i in range(P)]

    zn = jnp.zeros((NP,), jnp.float32)
    znd = jnp.zeros((N, D), jnp.float32)

    fq = jnp.concatenate([feat, query], axis=1)
    gathered = [_sc_gather(feat, fq, srcp[i], dstp[i], PARTS[i])
                for i in range(P)]

    ps, acc_views, sps = [], [], []
    for i in range(P):
        fu, fvqd = gathered[i]
        p, g = _tc_edge(OFFS[i], PARTS[i], fu, fvqd,
                        src_key_w, dst_key_w, edge_key_w,
                        src_key_b, dst_key_b, edge_key_b,
                        key_norm_w, key_norm_b,
                        src_val_w, dst_val_w, edge_val_w,
                        src_val_b, dst_val_b, edge_val_b,
                        value_norm_w, value_norm_b)
        p_flat = p.reshape(PARTS[i])
        acc, sp = _sc_scatter(g, p_flat, dstp[i],
                              dst[OFFS[i]:OFFS[i] + PARTS[i]], zn, znd,
                              PARTS[i])
        ps.append(p_flat)
        acc_views.extend([acc[:N], acc[N:]])
        sps.append(sp)

    p_all = jnp.concatenate(ps)
    attn = _sc_div(p_all, dst, sps)
    s_total = _tc_sred(sps).reshape(NP, 1)

    out = _tc_node(acc_views, s_total[:N],
                   node_weight[:, 0], node_bias[:, 0],
                   node_weight[:, 1], node_bias[:, 1],
                   node_norm_w, node_norm_b)
    return out, attn.reshape(E, 1)
